# baseline (speedup 1.0000x reference)
import functools

import jax
import jax.numpy as jnp
from jax.experimental import pallas as pl


def _silu(x):
    return x * jax.nn.sigmoid(x)


def _proj_kernel(x_ref, g1_ref, b1_ref, wbig_ref, bbig_ref, a2m_ref, a2b_ref,
                 s2m_ref, s2b_ref, hm_ref, qn_ref, kn_ref, val_ref, sc_ref,
                 *, d_q, d_a):
    xb = x_ref[...]
    mu = jnp.mean(xb, axis=-1, keepdims=True)
    var = jnp.mean((xb - mu) ** 2, axis=-1, keepdims=True)
    h = (xb - mu) * jax.lax.rsqrt(var + 1e-5) * g1_ref[...] + b1_ref[...]
    y = jnp.dot(h, wbig_ref[...]) + bbig_ref[...]
    q = _silu(y[:, 0:d_q])
    k = _silu(y[:, d_q:2 * d_q])
    v = _silu(y[:, 2 * d_q:3 * d_q])
    be = y[:, 3 * d_q:4 * d_q]
    a = y[:, 4 * d_q:4 * d_q + d_a]
    s1 = y[:, 4 * d_q + d_a:4 * d_q + 2 * d_a]
    hm = hm_ref[...]
    qs = jnp.dot(jnp.dot(q * q, hm), hm.T)
    ks = jnp.dot(jnp.dot(k * k, hm), hm.T)
    qn_ref[...] = q / jnp.maximum(jnp.sqrt(qs), 1e-12)
    kn_ref[...] = k / jnp.maximum(jnp.sqrt(ks), 1e-12)
    alpha = jax.nn.sigmoid(jnp.dot(a, a2m_ref[...]) + a2b_ref[...])
    beta_g = jax.nn.sigmoid(be)
    val_ref[...] = v * alpha + beta_g
    sc_ref[...] = jax.nn.sigmoid(jnp.dot(s1, s2m_ref[...]) + s2b_ref[...])


def _attn_kernel(qn_ref, kn_ref, val_ref, sc_ref, rms_ref, l1w_ref, l1b_ref,
                 cat_ref, *, n_h, hd):
    scale = 1.0 / (hd ** 0.5)
    for h in range(n_h):
        sl = slice(h * hd, (h + 1) * hd)
        qb = qn_ref[:, sl]
        kh = kn_ref[:, sl]
        vh = val_ref[:, sl]
        s = jax.lax.dot_general(qb, kh, (((1,), (1,)), ((), ()))) * scale
        m = jnp.max(s, axis=-1, keepdims=True)
        e = jnp.exp(s - m)
        w = e / jnp.sum(e, axis=-1, keepdims=True)
        attn = jnp.dot(w, vh)
        attn = attn * jax.lax.rsqrt(
            jnp.mean(attn * attn, axis=-1, keepdims=True) + 1e-6) * rms_ref[h]
        oh = attn * sc_ref[:, sl]
        cat_ref[:, sl] = jnp.dot(oh, l1w_ref[h]) + l1b_ref[h]


def _post_kernel(x_ref, cat_ref, ow_ref, ob_ref, g2_ref, b2_ref, rd_ref,
                 ru_ref, x1_ref, xf_ref, base_ref, log_ref):
    x1 = x_ref[...] + jnp.dot(cat_ref[...], ow_ref[...]) + ob_ref[...]
    x1_ref[...] = x1
    mu = jnp.mean(x1, axis=-1, keepdims=True)
    var = jnp.mean((x1 - mu) ** 2, axis=-1, keepdims=True)
    h2 = (x1 - mu) * jax.lax.rsqrt(var + 1e-5) * g2_ref[...] + b2_ref[...]
    xf_ref[...] = h2
    base_ref[...] = h2 * jax.lax.rsqrt(jnp.mean(h2 * h2, axis=-1,
                                                keepdims=True) + 1e-6)
    log_ref[...] = jnp.dot(jnp.dot(h2, rd_ref[...]), ru_ref[...])


def _route_kernel(log_ref, coeff_ref, lb_ref, *, n_e):
    lg = log_ref[...]
    t = lg.shape[0]
    idx = jax.lax.broadcasted_iota(jnp.int32, (t, n_e), 1)
    m1 = jnp.max(lg, axis=-1, keepdims=True)
    i1 = jnp.min(jnp.where(lg == m1, idx, n_e), axis=-1, keepdims=True)
    masked = jnp.where(idx == i1, -jnp.inf, lg)
    m2 = jnp.max(masked, axis=-1, keepdims=True)
    i2 = jnp.min(jnp.where(masked == m2, idx, n_e), axis=-1, keepdims=True)
    b = jnp.exp(m2 - m1)
    w1 = 1.0 / (1.0 + b)
    w2 = b / (1.0 + b)
    oh1 = (idx == i1).astype(jnp.float32)
    oh2 = (idx == i2).astype(jnp.float32)
    coeff_ref[...] = oh1 * w1 + oh2 * w2
    counts = jnp.sum(oh1 + oh2, axis=0)
    cm = jnp.mean(counts)
    lb_ref[...] = jnp.reshape(jnp.sum((counts - cm) ** 2) / (n_e - 1), (1, 1))


def _shared_kernel(base_ref, xf_ref, nw_ref, w1_ref, w2_ref, w3_ref, out_ref,
                   *, n_s):
    n = pl.program_id(1)
    xn = base_ref[...] * nw_ref[0]
    g = _silu(jnp.dot(xn, w1_ref[0]))
    v = jnp.dot(xn, w3_ref[0])
    t = jnp.dot(g * v, w2_ref[0]) + xf_ref[...]

    @pl.when(n == 0)
    def _():
        out_ref[...] = t

    @pl.when(n != 0)
    def _():
        out_ref[...] += t

    @pl.when(n == n_s - 1)
    def _():
        out_ref[...] *= 1.0 / n_s


def _moe_kernel(xf_ref, coeff_ref, w1_ref, w2_ref, w3_ref, x1_ref, sh_ref,
                out_ref, *, n_e):
    e = pl.program_id(1)
    xb = xf_ref[...]
    g = _silu(jnp.dot(xb, w1_ref[0]))
    v = jnp.dot(xb, w3_ref[0])
    t = jnp.dot(g * v, w2_ref[0])
    cb = coeff_ref[...]
    onehot = (jax.lax.broadcasted_iota(jnp.int32, cb.shape, 1) == e
              ).astype(jnp.float32)
    c = jnp.sum(cb * onehot, axis=1, keepdims=True)

    @pl.when(e == 0)
    def _():
        out_ref[...] = x1_ref[...] + sh_ref[...] + t * c

    @pl.when(e != 0)
    def _():
        out_ref[...] += t * c


def kernel(x, gamma1, beta1, gamma2, beta2, q_w, q_b, k_w, k_b, v_w, v_b,
           a1_w, a1_b, a2_w, a2_b, be_w, be_b, s1_w, s1_b, s2_w, s2_b,
           l1_w, l1_b, rms_w, out_w, out_b, sh_norm_w, sh_w1, sh_w2, sh_w3,
           r_w1, r_w2, r_w3, rd_w, ru_w):
    b, s, d = x.shape
    h_, _, hd = q_w.shape
    hh = a1_w.shape[2]
    ns, _, fs = sh_w1.shape
    n_e, _, fr = r_w1.shape
    d_q = h_ * hd
    d_a = h_ * hh
    f32 = jnp.float32

    x2 = x.reshape(s, d)

    def heads_to_cols(w):
        return jnp.transpose(w, (1, 0, 2)).reshape(d, -1)

    wbig = jnp.concatenate([heads_to_cols(q_w), heads_to_cols(k_w),
                            heads_to_cols(v_w), heads_to_cols(be_w),
                            heads_to_cols(a1_w), heads_to_cols(s1_w)], axis=1)
    bbig = jnp.concatenate([q_b.reshape(-1), k_b.reshape(-1),
                            v_b.reshape(-1), be_b.reshape(-1),
                            a1_b.reshape(-1), s1_b.reshape(-1)])[None, :]

    def block_diag(w):
        eye = jnp.eye(h_, dtype=f32)
        big = eye[:, None, :, None] * w[:, :, None, :]
        return big.reshape(h_ * hh, h_ * hd)

    a2m = block_diag(a2_w)
    s2m = block_diag(s2_w)
    a2b = a2_b.reshape(1, d_q)
    s2b = s2_b.reshape(1, d_q)

    head_mask = (jax.lax.broadcasted_iota(jnp.int32, (d_q, h_), 0) // hd ==
                 jax.lax.broadcasted_iota(jnp.int32, (d_q, h_), 1)
                 ).astype(f32)

    bs = 256
    nsb = s // bs

    qn, kn, val, sc = pl.pallas_call(
        functools.partial(_proj_kernel, d_q=d_q, d_a=d_a),
        grid=(nsb,),
        in_specs=[
            pl.BlockSpec((bs, d), lambda i: (i, 0)),
            pl.BlockSpec((1, d), lambda i: (0, 0)),
            pl.BlockSpec((1, d), lambda i: (0, 0)),
            pl.BlockSpec((d, 4 * d_q + 2 * d_a), lambda i: (0, 0)),
            pl.BlockSpec((1, 4 * d_q + 2 * d_a), lambda i: (0, 0)),
            pl.BlockSpec((d_a, d_q), lambda i: (0, 0)),
            pl.BlockSpec((1, d_q), lambda i: (0, 0)),
            pl.BlockSpec((d_a, d_q), lambda i: (0, 0)),
            pl.BlockSpec((1, d_q), lambda i: (0, 0)),
            pl.BlockSpec((d_q, h_), lambda i: (0, 0)),
        ],
        out_specs=[pl.BlockSpec((bs, d_q), lambda i: (i, 0))] * 4,
        out_shape=[jax.ShapeDtypeStruct((s, d_q), f32)] * 4,
    )(x2, gamma1[None, :], beta1[None, :], wbig, bbig, a2m, a2b, s2m, s2b,
      head_mask)

    cat = pl.pallas_call(
        functools.partial(_attn_kernel, n_h=h_, hd=hd),
        grid=(nsb,),
        in_specs=[
            pl.BlockSpec((bs, d_q), lambda i: (i, 0)),
            pl.BlockSpec((s, d_q), lambda i: (0, 0)),
            pl.BlockSpec((s, d_q), lambda i: (0, 0)),
            pl.BlockSpec((bs, d_q), lambda i: (i, 0)),
            pl.BlockSpec((h_, 1, hd), lambda i: (0, 0, 0)),
            pl.BlockSpec((h_, hd, hd), lambda i: (0, 0, 0)),
            pl.BlockSpec((h_, 1, hd), lambda i: (0, 0, 0)),
        ],
        out_specs=pl.BlockSpec((bs, d_q), lambda i: (i, 0)),
        out_shape=jax.ShapeDtypeStruct((s, d_q), f32),
    )(qn, kn, val, sc, rms_w.reshape(h_, 1, hd), l1_w,
      l1_b.reshape(h_, 1, hd))

    x1, xf, base, logits = pl.pallas_call(
        _post_kernel,
        grid=(nsb,),
        in_specs=[
            pl.BlockSpec((bs, d), lambda i: (i, 0)),
            pl.BlockSpec((bs, d), lambda i: (i, 0)),
            pl.BlockSpec((d, d), lambda i: (0, 0)),
            pl.BlockSpec((1, d), lambda i: (0, 0)),
            pl.BlockSpec((1, d), lambda i: (0, 0)),
            pl.BlockSpec((1, d), lambda i: (0, 0)),
            pl.BlockSpec((d, rd_w.shape[1]), lambda i: (0, 0)),
            pl.BlockSpec((rd_w.shape[1], n_e), lambda i: (0, 0)),
        ],
        out_specs=[
            pl.BlockSpec((bs, d), lambda i: (i, 0)),
            pl.BlockSpec((bs, d), lambda i: (i, 0)),
            pl.BlockSpec((bs, d), lambda i: (i, 0)),
            pl.BlockSpec((bs, n_e), lambda i: (i, 0)),
        ],
        out_shape=[
            jax.ShapeDtypeStruct((s, d), f32),
            jax.ShapeDtypeStruct((s, d), f32),
            jax.ShapeDtypeStruct((s, d), f32),
            jax.ShapeDtypeStruct((s, n_e), f32),
        ],
    )(x2, cat, out_w, out_b[None, :], gamma2[None, :], beta2[None, :],
      rd_w, ru_w)

    coeff, lb = pl.pallas_call(
        functools.partial(_route_kernel, n_e=n_e),
        out_shape=[jax.ShapeDtypeStruct((s, n_e), f32),
                   jax.ShapeDtypeStruct((1, 1), f32)],
    )(logits)

    shared = pl.pallas_call(
        functools.partial(_shared_kernel, n_s=ns),
        grid=(nsb, ns),
        in_specs=[
            pl.BlockSpec((bs, d), lambda i, n: (i, 0)),
            pl.BlockSpec((bs, d), lambda i, n: (i, 0)),
            pl.BlockSpec((1, 1, d), lambda i, n: (n, 0, 0)),
            pl.BlockSpec((1, d, fs), lambda i, n: (n, 0, 0)),
            pl.BlockSpec((1, fs, d), lambda i, n: (n, 0, 0)),
            pl.BlockSpec((1, d, fs), lambda i, n: (n, 0, 0)),
        ],
        out_specs=pl.BlockSpec((bs, d), lambda i, n: (i, 0)),
        out_shape=jax.ShapeDtypeStruct((s, d), f32),
    )(base, xf, sh_norm_w.reshape(ns, 1, d), sh_w1, sh_w2, sh_w3)

    out = pl.pallas_call(
        functools.partial(_moe_kernel, n_e=n_e),
        grid=(nsb, n_e),
        in_specs=[
            pl.BlockSpec((bs, d), lambda i, e: (i, 0)),
            pl.BlockSpec((bs, n_e), lambda i, e: (i, 0)),
            pl.BlockSpec((1, d, fr), lambda i, e: (e, 0, 0)),
            pl.BlockSpec((1, fr, d), lambda i, e: (e, 0, 0)),
            pl.BlockSpec((1, d, fr), lambda i, e: (e, 0, 0)),
            pl.BlockSpec((bs, d), lambda i, e: (i, 0)),
            pl.BlockSpec((bs, d), lambda i, e: (i, 0)),
        ],
        out_specs=pl.BlockSpec((bs, d), lambda i, e: (i, 0)),
        out_shape=jax.ShapeDtypeStruct((s, d), f32),
    )(xf, coeff, r_w1, r_w2, r_w3, x1, shared)

    return out.reshape(b, s, d), lb.reshape(())
